# baseline (speedup 1.0000x reference)
import functools

import jax
import jax.numpy as jnp
from jax.experimental import pallas as pl
from jax.experimental.pallas import tpu as pltpu

POOL = 2
VMEM_LIMIT = 48 * 1024 * 1024
G_CONV1 = 1
G_CONV2 = 2


def _sig(x):
    return pl.reciprocal(1.0 + jnp.exp(-x), approx=False)


def _conv_body(G, hp, kh, N, split_c, x_ref, t_ref, b_ref, o_ref):
    if split_c:
        C = x_ref.shape[1]
        planes = [jnp.concatenate(
            [x_ref[g, c].astype(jnp.bfloat16) for c in range(C)], axis=1)
            for g in range(G)]
    else:
        planes = [x_ref[g] for g in range(G)]
    mr = POOL * hp
    acc = None
    for i in range(kh):
        rows = [planes[g][i: i + mr] for g in range(G)]
        band = rows[0] if G == 1 else jnp.concatenate(rows, axis=0)
        d = jnp.dot(band, t_ref[i], preferred_element_type=jnp.float32)
        acc = d if acc is None else acc + d
    mo = G * hp
    mi = G * mr
    row_o = jax.lax.broadcasted_iota(jnp.int32, (mo, mi), 0)
    row_i = jax.lax.broadcasted_iota(jnp.int32, (mo, mi), 1)
    base = 2 * row_o
    s_even = (row_i == base).astype(jnp.bfloat16)
    s_odd = (row_i == base + 1).astype(jnp.bfloat16)
    acc_hi = acc.astype(jnp.bfloat16)
    acc_lo = (acc - acc_hi.astype(jnp.float32)).astype(jnp.bfloat16)
    even = (jnp.dot(s_even, acc_hi, preferred_element_type=jnp.float32)
            + jnp.dot(s_even, acc_lo, preferred_element_type=jnp.float32))
    odd = (jnp.dot(s_odd, acc_hi, preferred_element_type=jnp.float32)
           + jnp.dot(s_odd, acc_lo, preferred_element_type=jnp.float32))
    m = jnp.maximum(even, odd)
    m = jnp.maximum(m[:, :N], m[:, N:])
    o_ref[...] = _sig(m + b_ref[...]).astype(o_ref.dtype).reshape(G, hp, N)


def _conv_stage(x_in, t_cat, b_row, G):
    split_c = x_in.ndim == 4
    B = x_in.shape[0]
    kh, WC, N2 = t_cat.shape
    N = N2 // 2
    H = x_in.shape[2] if split_c else x_in.shape[1]
    hp = (H - kh + 1) // POOL
    if split_c:
        in_spec = pl.BlockSpec((G,) + x_in.shape[1:], lambda i: (i, 0, 0, 0))
    else:
        in_spec = pl.BlockSpec((G, H, WC), lambda i: (i, 0, 0))
    return pl.pallas_call(
        functools.partial(_conv_body, G, hp, kh, N, split_c),
        out_shape=jax.ShapeDtypeStruct((B, hp, N), jnp.bfloat16),
        grid=(B // G,),
        in_specs=[
            in_spec,
            pl.BlockSpec((kh, WC, N2), lambda i: (0, 0, 0)),
            pl.BlockSpec((1, N), lambda i: (0, 0)),
        ],
        out_specs=pl.BlockSpec((G, hp, N), lambda i: (i, 0, 0)),
        compiler_params=pltpu.CompilerParams(
            dimension_semantics=("parallel",),
            vmem_limit_bytes=VMEM_LIMIT),
    )(x_in, t_cat, b_row)


def _fc_body(x_ref, w1_ref, b1_ref, w2_ref, b2_ref, w3_ref, b3_ref, o_ref):
    R = x_ref.shape[1]
    acc = None
    for r in range(R):
        d = jnp.dot(x_ref[:, r, :], w1_ref[r].astype(jnp.bfloat16),
                    preferred_element_type=jnp.float32)
        acc = d if acc is None else acc + d
    h1 = _sig(acc + b1_ref[...])
    h2 = _sig(jnp.dot(h1, w2_ref[...],
                      preferred_element_type=jnp.float32) + b2_ref[...])
    o_ref[...] = (jnp.dot(h2, w3_ref[...],
                          preferred_element_type=jnp.float32) + b3_ref[...])


def _fc_stage(y2, w1, b1, w2, b2, w3, b3):
    MB, R, NF = y2.shape
    H1, H2, NC = w1.shape[2], w2.shape[1], w3.shape[1]
    MT = MB // 2 if MB % 16 == 0 else MB
    return pl.pallas_call(
        _fc_body,
        out_shape=jax.ShapeDtypeStruct((MB, NC), jnp.float32),
        grid=(MB // MT,),
        in_specs=[
            pl.BlockSpec((MT, R, NF), lambda i: (i, 0, 0)),
            pl.BlockSpec((R, NF, H1), lambda i: (0, 0, 0)),
            pl.BlockSpec((1, H1), lambda i: (0, 0)),
            pl.BlockSpec((H1, H2), lambda i: (0, 0)),
            pl.BlockSpec((1, H2), lambda i: (0, 0)),
            pl.BlockSpec((H2, NC), lambda i: (0, 0)),
            pl.BlockSpec((1, NC), lambda i: (0, 0)),
        ],
        out_specs=pl.BlockSpec((MT, NC), lambda i: (i, 0)),
        compiler_params=pltpu.CompilerParams(
            dimension_semantics=("parallel",),
            vmem_limit_bytes=VMEM_LIMIT),
    )(y2, w1, b1.reshape(1, H1), w2, b2.reshape(1, H2), w3, b3.reshape(1, NC))


def _prep2_body(kh, t_ref, o_ref):
    for i in range(kh):
        o_ref[i] = jnp.concatenate(
            [t_ref[0, i].astype(jnp.bfloat16),
             t_ref[1, i].astype(jnp.bfloat16)], axis=1)


def _prep2(t):
    kh, WC, N = t.shape[1], t.shape[2], t.shape[3]
    return pl.pallas_call(
        functools.partial(_prep2_body, kh),
        out_shape=jax.ShapeDtypeStruct((kh, WC, 2 * N), jnp.bfloat16),
        compiler_params=pltpu.CompilerParams(
            vmem_limit_bytes=VMEM_LIMIT),
    )(t)


def _prep1_body(kh, W, C, t_ref, o_ref):
    WC = W * C
    r_out = jax.lax.broadcasted_iota(jnp.int32, (WC, WC), 0)
    r_in = jax.lax.broadcasted_iota(jnp.int32, (WC, WC), 1)
    perm = ((r_out % W) * C + r_out // W == r_in).astype(jnp.bfloat16)
    for i in range(kh):
        pb = [jnp.dot(perm, t_ref[dw, i].astype(jnp.bfloat16),
                      preferred_element_type=jnp.float32).astype(jnp.bfloat16)
              for dw in range(2)]
        o_ref[i] = jnp.concatenate(pb, axis=1)


def _prep1(t, W, C):
    kh, WC, N = t.shape[1], t.shape[2], t.shape[3]
    return pl.pallas_call(
        functools.partial(_prep1_body, kh, W, C),
        out_shape=jax.ShapeDtypeStruct((kh, WC, 2 * N), jnp.bfloat16),
        compiler_params=pltpu.CompilerParams(
            vmem_limit_bytes=VMEM_LIMIT),
    )(t)


def kernel(x, t1, b1, t2, b2, fc1_w, fc1_b, fc2_w, fc2_b, fc3_w, fc3_b):
    B, C, H, W = x.shape

    y1 = _conv_stage(x, _prep1(t1, W, C), b1, G_CONV1)
    y2 = _conv_stage(y1, _prep2(t2), b2, G_CONV2)

    w1 = fc1_w.reshape(y2.shape[1], y2.shape[2], fc1_w.shape[1])
    return _fc_stage(y2, w1, fc1_b, fc2_w, fc2_b, fc3_w, fc3_b)


# speedup vs baseline: 1.0886x; 1.0886x over previous
import functools

import jax
import jax.numpy as jnp
from jax.experimental import pallas as pl
from jax.experimental.pallas import tpu as pltpu

POOL = 2
VMEM_LIMIT = 48 * 1024 * 1024
G_CONV1 = 1
G_CONV2 = 2


def _sig(x):
    return pl.reciprocal(1.0 + jnp.exp(-x), approx=False)


def _conv_body(G, hp, kh, N, split_c, x_ref, t_ref, b_ref, o_ref):
    if split_c:
        C = x_ref.shape[1]
        planes = [jnp.concatenate(
            [x_ref[g, c].astype(jnp.bfloat16) for c in range(C)], axis=1)
            for g in range(G)]
    else:
        planes = [x_ref[g] for g in range(G)]
    mr = POOL * hp
    acc = None
    for i in range(kh):
        rows = [planes[g][i: i + mr] for g in range(G)]
        band = rows[0] if G == 1 else jnp.concatenate(rows, axis=0)
        d = jnp.dot(band, t_ref[i], preferred_element_type=jnp.float32)
        acc = d if acc is None else acc + d
    z = _sig(jnp.maximum(acc[:, :N], acc[:, N:]) + b_ref[...])
    zb = z.astype(jnp.bfloat16)
    mo = G * hp
    mi = G * mr
    row_o = jax.lax.broadcasted_iota(jnp.int32, (2 * mo, mi), 0)
    row_i = jax.lax.broadcasted_iota(jnp.int32, (2 * mo, mi), 1)
    sel = (row_i == 2 * (row_o % mo) + row_o // mo).astype(jnp.bfloat16)
    eo = jnp.dot(sel, zb, preferred_element_type=jnp.float32)
    m = jnp.maximum(eo[:mo], eo[mo:])
    o_ref[...] = m.astype(o_ref.dtype).reshape(G, hp, N)


def _conv_stage(x_in, t_cat, b_row, G):
    split_c = x_in.ndim == 4
    B = x_in.shape[0]
    kh, WC, N2 = t_cat.shape
    N = N2 // 2
    H = x_in.shape[2] if split_c else x_in.shape[1]
    hp = (H - kh + 1) // POOL
    if split_c:
        in_spec = pl.BlockSpec((G,) + x_in.shape[1:], lambda i: (i, 0, 0, 0))
    else:
        in_spec = pl.BlockSpec((G, H, WC), lambda i: (i, 0, 0))
    return pl.pallas_call(
        functools.partial(_conv_body, G, hp, kh, N, split_c),
        out_shape=jax.ShapeDtypeStruct((B, hp, N), jnp.bfloat16),
        grid=(B // G,),
        in_specs=[
            in_spec,
            pl.BlockSpec((kh, WC, N2), lambda i: (0, 0, 0)),
            pl.BlockSpec((1, N), lambda i: (0, 0)),
        ],
        out_specs=pl.BlockSpec((G, hp, N), lambda i: (i, 0, 0)),
        compiler_params=pltpu.CompilerParams(
            dimension_semantics=("parallel",),
            vmem_limit_bytes=VMEM_LIMIT),
    )(x_in, t_cat, b_row)


def _fc_body(x_ref, w1_ref, b1_ref, w2_ref, b2_ref, w3_ref, b3_ref, o_ref):
    R, NF = x_ref.shape[1], x_ref.shape[2]
    acc = None
    for r in range(R):
        d = jnp.dot(x_ref[:, r, :],
                    w1_ref[r * NF:(r + 1) * NF, :].astype(jnp.bfloat16),
                    preferred_element_type=jnp.float32)
        acc = d if acc is None else acc + d
    h1 = _sig(acc + b1_ref[...])
    h2 = _sig(jnp.dot(h1, w2_ref[...],
                      preferred_element_type=jnp.float32) + b2_ref[...])
    o_ref[...] = (jnp.dot(h2, w3_ref[...],
                          preferred_element_type=jnp.float32) + b3_ref[...])


def _fc_stage(y2, w1, b1, w2, b2, w3, b3):
    MB, R, NF = y2.shape
    H1, H2, NC = w1.shape[1], w2.shape[1], w3.shape[1]
    MT = MB // 2 if MB % 16 == 0 else MB
    return pl.pallas_call(
        _fc_body,
        out_shape=jax.ShapeDtypeStruct((MB, NC), jnp.float32),
        grid=(MB // MT,),
        in_specs=[
            pl.BlockSpec((MT, R, NF), lambda i: (i, 0, 0)),
            pl.BlockSpec((R * NF, H1), lambda i: (0, 0)),
            pl.BlockSpec((1, H1), lambda i: (0, 0)),
            pl.BlockSpec((H1, H2), lambda i: (0, 0)),
            pl.BlockSpec((1, H2), lambda i: (0, 0)),
            pl.BlockSpec((H2, NC), lambda i: (0, 0)),
            pl.BlockSpec((1, NC), lambda i: (0, 0)),
        ],
        out_specs=pl.BlockSpec((MT, NC), lambda i: (i, 0)),
        compiler_params=pltpu.CompilerParams(
            dimension_semantics=("parallel",),
            vmem_limit_bytes=VMEM_LIMIT),
    )(y2, w1, b1.reshape(1, H1), w2, b2.reshape(1, H2), w3, b3.reshape(1, NC))


def _prep2_body(kh, t_ref, o_ref):
    for i in range(kh):
        o_ref[i] = jnp.concatenate(
            [t_ref[0, i].astype(jnp.bfloat16),
             t_ref[1, i].astype(jnp.bfloat16)], axis=1)


def _prep2(t):
    kh, WC, N = t.shape[1], t.shape[2], t.shape[3]
    return pl.pallas_call(
        functools.partial(_prep2_body, kh),
        out_shape=jax.ShapeDtypeStruct((kh, WC, 2 * N), jnp.bfloat16),
        compiler_params=pltpu.CompilerParams(
            vmem_limit_bytes=VMEM_LIMIT),
    )(t)


def _prep1_body(kh, W, C, t_ref, o_ref):
    WC = W * C
    r_out = jax.lax.broadcasted_iota(jnp.int32, (WC, WC), 0)
    r_in = jax.lax.broadcasted_iota(jnp.int32, (WC, WC), 1)
    perm = ((r_out % W) * C + r_out // W == r_in).astype(jnp.bfloat16)
    for i in range(kh):
        pb = [jnp.dot(perm, t_ref[dw, i].astype(jnp.bfloat16),
                      preferred_element_type=jnp.float32).astype(jnp.bfloat16)
              for dw in range(2)]
        o_ref[i] = jnp.concatenate(pb, axis=1)


def _prep1(t, W, C):
    kh, WC, N = t.shape[1], t.shape[2], t.shape[3]
    return pl.pallas_call(
        functools.partial(_prep1_body, kh, W, C),
        out_shape=jax.ShapeDtypeStruct((kh, WC, 2 * N), jnp.bfloat16),
        compiler_params=pltpu.CompilerParams(
            vmem_limit_bytes=VMEM_LIMIT),
    )(t)


def kernel(x, t1, b1, t2, b2, fc1_w, fc1_b, fc2_w, fc2_b, fc3_w, fc3_b):
    B, C, H, W = x.shape

    y1 = _conv_stage(x, _prep1(t1, W, C), b1, G_CONV1)
    y2 = _conv_stage(y1, _prep2(t2), b2, G_CONV2)

    return _fc_stage(y2, fc1_w, fc1_b, fc2_w, fc2_b, fc3_w, fc3_b)


# speedup vs baseline: 1.2560x; 1.1538x over previous
import functools

import jax
import jax.numpy as jnp
from jax.experimental import pallas as pl
from jax.experimental.pallas import tpu as pltpu

POOL = 2
VMEM_LIMIT = 48 * 1024 * 1024
G_CONV1 = 2
G_CONV2 = 4


def _sig(x):
    return pl.reciprocal(1.0 + jnp.exp(-x), approx=True)


def _conv_body(G, hp, kh, N, split_c, x_ref, t_ref, b_ref, o_ref):
    if split_c:
        C = x_ref.shape[0] // G
        planes = [jnp.concatenate(
            [x_ref[g * C + c].astype(jnp.bfloat16) for c in range(C)], axis=1)
            for g in range(G)]
    else:
        planes = [x_ref[g] for g in range(G)]
    mr = POOL * hp
    acc = None
    for i in range(kh):
        rows = [planes[g][i: i + mr] for g in range(G)]
        band = rows[0] if G == 1 else jnp.concatenate(rows, axis=0)
        d = jnp.dot(band, t_ref[i], preferred_element_type=jnp.float32)
        acc = d if acc is None else acc + d
    z = _sig(jnp.maximum(acc[:, :N], acc[:, N:]) + b_ref[...])
    zb = z.astype(jnp.bfloat16)
    mo = G * hp
    mi = G * mr
    row_o = jax.lax.broadcasted_iota(jnp.int32, (2 * mo, mi), 0)
    row_i = jax.lax.broadcasted_iota(jnp.int32, (2 * mo, mi), 1)
    sel = (row_i == 2 * (row_o % mo) + row_o // mo).astype(jnp.bfloat16)
    eo = jnp.dot(sel, zb, preferred_element_type=jnp.float32)
    m = jnp.maximum(eo[:mo], eo[mo:])
    o_ref[...] = m.astype(o_ref.dtype).reshape(G, hp, N)


def _conv_stage(x_in, t_cat, b_row, G, split_c=False, B=None):
    if not split_c:
        B = x_in.shape[0]
    C = x_in.shape[0] // B
    kh, WC, N2 = t_cat.shape
    N = N2 // 2
    H = x_in.shape[1]
    hp = (H - kh + 1) // POOL
    if split_c:
        in_spec = pl.BlockSpec((G * C, H, x_in.shape[2]), lambda i: (i, 0, 0))
    else:
        in_spec = pl.BlockSpec((G, H, WC), lambda i: (i, 0, 0))
    return pl.pallas_call(
        functools.partial(_conv_body, G, hp, kh, N, split_c),
        out_shape=jax.ShapeDtypeStruct((B, hp, N), jnp.bfloat16),
        grid=(B // G,),
        in_specs=[
            in_spec,
            pl.BlockSpec((kh, WC, N2), lambda i: (0, 0, 0)),
            pl.BlockSpec((1, N), lambda i: (0, 0)),
        ],
        out_specs=pl.BlockSpec((G, hp, N), lambda i: (i, 0, 0)),
        compiler_params=pltpu.CompilerParams(
            dimension_semantics=("parallel",),
            vmem_limit_bytes=VMEM_LIMIT),
    )(x_in, t_cat, b_row)


def _fc_body(x_ref, w1_ref, b1_ref, w2_ref, b2_ref, w3_ref, b3_ref, o_ref):
    R, NF = x_ref.shape[1], x_ref.shape[2]
    acc = None
    for r in range(R):
        d = jnp.dot(x_ref[:, r, :], w1_ref[r * NF:(r + 1) * NF, :],
                    preferred_element_type=jnp.float32)
        acc = d if acc is None else acc + d
    h1 = _sig(acc + b1_ref[...])
    h2 = _sig(jnp.dot(h1, w2_ref[...],
                      preferred_element_type=jnp.float32) + b2_ref[...])
    o_ref[...] = (jnp.dot(h2, w3_ref[...],
                          preferred_element_type=jnp.float32) + b3_ref[...])


def _fc_stage(y2, w1, b1, w2, b2, w3, b3):
    MB, R, NF = y2.shape
    H1, H2, NC = w1.shape[1], w2.shape[1], w3.shape[1]
    MT = MB // 2 if MB % 16 == 0 else MB
    return pl.pallas_call(
        _fc_body,
        out_shape=jax.ShapeDtypeStruct((MB, NC), jnp.float32),
        grid=(MB // MT,),
        in_specs=[
            pl.BlockSpec((MT, R, NF), lambda i: (i, 0, 0)),
            pl.BlockSpec((R * NF, H1), lambda i: (0, 0)),
            pl.BlockSpec((1, H1), lambda i: (0, 0)),
            pl.BlockSpec((H1, H2), lambda i: (0, 0)),
            pl.BlockSpec((1, H2), lambda i: (0, 0)),
            pl.BlockSpec((H2, NC), lambda i: (0, 0)),
            pl.BlockSpec((1, NC), lambda i: (0, 0)),
        ],
        out_specs=pl.BlockSpec((MT, NC), lambda i: (i, 0)),
        compiler_params=pltpu.CompilerParams(
            dimension_semantics=("parallel",),
            vmem_limit_bytes=VMEM_LIMIT),
    )(y2, w1, b1.reshape(1, H1), w2, b2.reshape(1, H2), w3, b3.reshape(1, NC))


def _prep2_body(kh, t_ref, o_ref):
    for i in range(kh):
        o_ref[i] = jnp.concatenate(
            [t_ref[0, i].astype(jnp.bfloat16),
             t_ref[1, i].astype(jnp.bfloat16)], axis=1)


def _prep2(t):
    kh, WC, N = t.shape[1], t.shape[2], t.shape[3]
    return pl.pallas_call(
        functools.partial(_prep2_body, kh),
        out_shape=jax.ShapeDtypeStruct((kh, WC, 2 * N), jnp.bfloat16),
        compiler_params=pltpu.CompilerParams(
            vmem_limit_bytes=VMEM_LIMIT),
    )(t)


def _prep1_body(kh, W, C, t_ref, o_ref):
    WC = W * C
    r_out = jax.lax.broadcasted_iota(jnp.int32, (WC, WC), 0)
    r_in = jax.lax.broadcasted_iota(jnp.int32, (WC, WC), 1)
    perm = ((r_out % W) * C + r_out // W == r_in).astype(jnp.bfloat16)
    for i in range(kh):
        pb = [jnp.dot(perm, t_ref[dw, i].astype(jnp.bfloat16),
                      preferred_element_type=jnp.float32).astype(jnp.bfloat16)
              for dw in range(2)]
        o_ref[i] = jnp.concatenate(pb, axis=1)


def _prep1(t, W, C):
    kh, WC, N = t.shape[1], t.shape[2], t.shape[3]
    return pl.pallas_call(
        functools.partial(_prep1_body, kh, W, C),
        out_shape=jax.ShapeDtypeStruct((kh, WC, 2 * N), jnp.bfloat16),
        compiler_params=pltpu.CompilerParams(
            vmem_limit_bytes=VMEM_LIMIT),
    )(t)


def kernel(x, t1, b1, t2, b2, fc1_w, fc1_b, fc2_w, fc2_b, fc3_w, fc3_b):
    B, C, H, W = x.shape

    xp = x.reshape(B * C, H, W)
    y1 = _conv_stage(xp, _prep1(t1, W, C), b1, G_CONV1,
                     split_c=True, B=B)
    y2 = _conv_stage(y1, _prep2(t2), b2, G_CONV2)

    return _fc_stage(y2, fc1_w.astype(jnp.bfloat16), fc1_b,
                     fc2_w, fc2_b, fc3_w, fc3_b)


# speedup vs baseline: 1.4607x; 1.1630x over previous
import functools

import jax
import jax.numpy as jnp
from jax.experimental import pallas as pl
from jax.experimental.pallas import tpu as pltpu

POOL = 2
VMEM_LIMIT = 48 * 1024 * 1024
G_CONV1 = 2
G_CONV2 = 4
Q_CHUNK1 = 2
Q_CHUNK2 = 1


def _sig(x):
    return pl.reciprocal(1.0 + jnp.exp(-x), approx=True)


def _conv_body(G, hp, kh, Wp, cout, Q, split_c, x_ref, t_ref, b_ref, o_ref):
    N = Wp * cout
    P = Wp // Q
    Lc = 2 * P + kh - 1
    mr = POOL * hp
    nq = P * cout
    KP = t_ref.shape[2]
    if split_c:
        C = x_ref.shape[0] // G
        def plane_q(g, q):
            parts = [x_ref[g * C + c][:, 2 * P * q: 2 * P * q + Lc]
                     .astype(jnp.bfloat16) for c in range(C)]
            if KP > Lc * C:
                parts.append(jnp.zeros((x_ref.shape[1], KP - Lc * C),
                                       jnp.bfloat16))
            return jnp.concatenate(parts, axis=1)
    else:
        cin = x_ref.shape[2] // (2 * Wp + kh - 1)
        def plane_q(g, q):
            sl = x_ref[g][:, 2 * P * q * cin: (2 * P * q + Lc) * cin]
            if KP > Lc * cin:
                sl = jnp.concatenate(
                    [sl, jnp.zeros((x_ref.shape[1], KP - Lc * cin),
                                   jnp.bfloat16)], axis=1)
            return sl
    zs = []
    for q in range(Q):
        planes = [plane_q(g, q) for g in range(G)]
        acc = None
        for i in range(kh):
            rows = [planes[g][i: i + mr] for g in range(G)]
            band = rows[0] if G == 1 else jnp.concatenate(rows, axis=0)
            d = jnp.dot(band, t_ref[q, i], preferred_element_type=jnp.float32)
            acc = d if acc is None else acc + d
        v = jnp.maximum(acc[:, :nq], acc[:, nq:])
        zs.append(_sig(v + b_ref[:, nq * q: nq * (q + 1)]).astype(jnp.bfloat16))
    mo = G * hp
    mi = G * mr
    row_o = jax.lax.broadcasted_iota(jnp.int32, (2 * mo, mi), 0)
    row_i = jax.lax.broadcasted_iota(jnp.int32, (2 * mo, mi), 1)
    sel = (row_i == 2 * (row_o % mo) + row_o // mo).astype(jnp.bfloat16)
    ms = []
    for zq in zs:
        eo = jnp.dot(sel, zq, preferred_element_type=jnp.float32)
        ms.append(jnp.maximum(eo[:mo], eo[mo:]))
    m = ms[0] if len(ms) == 1 else jnp.concatenate(ms, axis=1)
    o_ref[...] = m.astype(o_ref.dtype).reshape(G, hp, N)


def _conv_stage(x_in, t_cat, b_row, G, cout, split_c=False, B=None):
    if not split_c:
        B = x_in.shape[0]
    Q, kh, Kc, NQ2 = t_cat.shape
    Wp = Q * NQ2 // (2 * cout)
    N = Wp * cout
    H = x_in.shape[1]
    hp = (H - kh + 1) // POOL
    if split_c:
        C = x_in.shape[0] // B
        in_spec = pl.BlockSpec((G * C, H, x_in.shape[2]), lambda i: (i, 0, 0))
    else:
        in_spec = pl.BlockSpec((G, H, x_in.shape[2]), lambda i: (i, 0, 0))
    return pl.pallas_call(
        functools.partial(_conv_body, G, hp, kh, Wp, cout, Q, split_c),
        out_shape=jax.ShapeDtypeStruct((B, hp, N), jnp.bfloat16),
        grid=(B // G,),
        in_specs=[
            in_spec,
            pl.BlockSpec((Q, kh, Kc, NQ2), lambda i: (0, 0, 0, 0)),
            pl.BlockSpec((1, N), lambda i: (0, 0)),
        ],
        out_specs=pl.BlockSpec((G, hp, N), lambda i: (i, 0, 0)),
        compiler_params=pltpu.CompilerParams(
            dimension_semantics=("parallel",),
            vmem_limit_bytes=VMEM_LIMIT),
    )(x_in, t_cat, b_row)


def _fc_body(x_ref, w1_ref, b1_ref, w2_ref, b2_ref, w3_ref, b3_ref, o_ref):
    R, NF = x_ref.shape[1], x_ref.shape[2]
    acc = None
    for r in range(R):
        d = jnp.dot(x_ref[:, r, :], w1_ref[r * NF:(r + 1) * NF, :],
                    preferred_element_type=jnp.float32)
        acc = d if acc is None else acc + d
    h1 = _sig(acc + b1_ref[...])
    h2 = _sig(jnp.dot(h1, w2_ref[...],
                      preferred_element_type=jnp.float32) + b2_ref[...])
    o_ref[...] = (jnp.dot(h2, w3_ref[...],
                          preferred_element_type=jnp.float32) + b3_ref[...])


def _fc_stage(y2, w1, b1, w2, b2, w3, b3):
    MB, R, NF = y2.shape
    H1, H2, NC = w1.shape[1], w2.shape[1], w3.shape[1]
    MT = MB // 2 if MB % 16 == 0 else MB
    return pl.pallas_call(
        _fc_body,
        out_shape=jax.ShapeDtypeStruct((MB, NC), jnp.float32),
        grid=(MB // MT,),
        in_specs=[
            pl.BlockSpec((MT, R, NF), lambda i: (i, 0, 0)),
            pl.BlockSpec((R * NF, H1), lambda i: (0, 0)),
            pl.BlockSpec((1, H1), lambda i: (0, 0)),
            pl.BlockSpec((H1, H2), lambda i: (0, 0)),
            pl.BlockSpec((1, H2), lambda i: (0, 0)),
            pl.BlockSpec((H2, NC), lambda i: (0, 0)),
            pl.BlockSpec((1, NC), lambda i: (0, 0)),
        ],
        out_specs=pl.BlockSpec((MT, NC), lambda i: (i, 0)),
        compiler_params=pltpu.CompilerParams(
            dimension_semantics=("parallel",),
            vmem_limit_bytes=VMEM_LIMIT),
    )(y2, w1, b1.reshape(1, H1), w2, b2.reshape(1, H2), w3, b3.reshape(1, NC))


def _prep2_body(kh, Wp, cout, cin, Q, t_ref, o_ref):
    P = Wp // Q
    Lc = 2 * P + kh - 1
    nq = P * cout
    KP = o_ref.shape[2]
    zpad = (jnp.zeros((KP - Lc * cin, 2 * nq), jnp.bfloat16)
            if KP > Lc * cin else None)
    for q in range(Q):
        r0 = 2 * P * q * cin
        for i in range(kh):
            v = jnp.concatenate(
                [t_ref[dw, i][r0:r0 + Lc * cin, nq * q:nq * (q + 1)]
                 .astype(jnp.bfloat16) for dw in range(2)], axis=1)
            o_ref[q, i] = (v if zpad is None
                           else jnp.concatenate([v, zpad], axis=0))


def _prep2(t, Wp, cout, cin, Q):
    kh = t.shape[1]
    P = Wp // Q
    Lc = 2 * P + kh - 1
    return pl.pallas_call(
        functools.partial(_prep2_body, kh, Wp, cout, cin, Q),
        out_shape=jax.ShapeDtypeStruct(
            (Q, kh, ((Lc * cin + 255) // 256) * 256 if Q > 1 else Lc * cin,
             2 * P * cout), jnp.bfloat16),
        compiler_params=pltpu.CompilerParams(
            vmem_limit_bytes=VMEM_LIMIT),
    )(t)


def _prep1_body(kh, W, C, Wp, cout, Q, t_ref, o_ref):
    WC = W * C
    r_out = jax.lax.broadcasted_iota(jnp.int32, (WC, WC), 0)
    r_in = jax.lax.broadcasted_iota(jnp.int32, (WC, WC), 1)
    perm = ((r_out % W) * C + r_out // W == r_in).astype(jnp.bfloat16)
    pb = [[jnp.dot(perm, t_ref[dw, i].astype(jnp.bfloat16),
                   preferred_element_type=jnp.float32).astype(jnp.bfloat16)
           for i in range(kh)] for dw in range(2)]
    P = Wp // Q
    Lc = 2 * P + kh - 1
    nq = P * cout
    KP = o_ref.shape[2]
    zpad = (jnp.zeros((KP - Lc * C, 2 * nq), jnp.bfloat16)
            if KP > Lc * C else None)
    for q in range(Q):
        for i in range(kh):
            blocks = []
            for dw in range(2):
                rows = jnp.concatenate(
                    [pb[dw][i][c * W + 2 * P * q: c * W + 2 * P * q + Lc]
                     for c in range(C)], axis=0)
                blocks.append(rows[:, nq * q:nq * (q + 1)])
            v = jnp.concatenate(blocks, axis=1)
            o_ref[q, i] = (v if zpad is None
                           else jnp.concatenate([v, zpad], axis=0))


def _prep1(t, W, C, Wp, cout, Q):
    kh = t.shape[1]
    P = Wp // Q
    Lc = 2 * P + kh - 1
    return pl.pallas_call(
        functools.partial(_prep1_body, kh, W, C, Wp, cout, Q),
        out_shape=jax.ShapeDtypeStruct(
            (Q, kh, ((Lc * C + 255) // 256) * 256 if Q > 1 else Lc * C,
             2 * P * cout), jnp.bfloat16),
        compiler_params=pltpu.CompilerParams(
            vmem_limit_bytes=VMEM_LIMIT),
    )(t)


def kernel(x, t1, b1, t2, b2, fc1_w, fc1_b, fc2_w, fc2_b, fc3_w, fc3_b):
    B, C, H, W = x.shape
    kh = t1.shape[1]
    Wp1 = (W - kh + 1) // POOL
    cout1 = t1.shape[3] // Wp1
    kh2 = t2.shape[1]
    Wp2 = (Wp1 - kh2 + 1) // POOL
    cout2 = t2.shape[3] // Wp2

    xp = x.reshape(B * C, H, W)
    y1 = _conv_stage(xp, _prep1(t1, W, C, Wp1, cout1, Q_CHUNK1), b1, G_CONV1,
                     cout1, split_c=True, B=B)
    y2 = _conv_stage(y1, _prep2(t2, Wp2, cout2, cout1, Q_CHUNK2), b2, G_CONV2,
                     cout2)

    return _fc_stage(y2, fc1_w.astype(jnp.bfloat16), fc1_b,
                     fc2_w, fc2_b, fc3_w, fc3_b)
